# baseline (device time: 107494 ns/iter reference)
import jax
import jax.numpy as jnp
from jax import lax
from jax.experimental import pallas as pl
from jax.experimental.pallas import tpu as pltpu

N_DEV = 32
N_GRP = 16
B = 2
S = 128
HD = 256
D_OUT = 512


def kernel(x, Wq, K_ext, V_ext, Wo):
    K2 = K_ext.reshape(B, S, HD)
    V2 = V_ext.reshape(B, S, HD)

    def body(x_ref, wq_ref, k_ref, v_ref, wo_ref, out_ref,
             kv_all, q_scr, ctx_scr, send_sems, recv_sems):
        my = lax.axis_index("i")
        right = lax.rem(my + 2, N_DEV)
        left = lax.rem(my + N_DEV - 2, N_DEV)

        barrier_sem = pltpu.get_barrier_semaphore()
        for nbr in (left, right):
            pl.semaphore_signal(
                barrier_sem, inc=1,
                device_id=(nbr,), device_id_type=pl.DeviceIdType.MESH,
            )
        pl.semaphore_wait(barrier_sem, 2)

        kv_all[0, 0] = k_ref[...]
        kv_all[0, 1] = v_ref[...]

        xm = x_ref[...].reshape(B * S, D_OUT)
        q = jnp.dot(xm, wq_ref[...], preferred_element_type=jnp.float32)
        q_scr[...] = (q * 0.125).reshape(B, S, HD)

        chains = [(kv, b) for kv in range(2) for b in range(B)]
        rdmas = [[] for _ in chains]
        for h in range(N_GRP - 1):
            for c, (kv, b) in enumerate(chains):
                if h > 0:
                    rdmas[c][h - 1].wait_recv()
                r = pltpu.make_async_remote_copy(
                    src_ref=kv_all.at[h, kv, b],
                    dst_ref=kv_all.at[h + 1, kv, b],
                    send_sem=send_sems.at[c, h],
                    recv_sem=recv_sems.at[c, h],
                    device_id=(right,),
                    device_id_type=pl.DeviceIdType.MESH,
                )
                r.start()
                rdmas[c].append(r)
        for c in range(len(chains)):
            rdmas[c][-1].wait_recv()
        for h in range(N_GRP - 1):
            for c in range(len(chains)):
                rdmas[c][h].wait_send()

        for b in range(B):
            for hh in range(4):
                for blk in range(2):
                    rows = pl.ds(blk * 64, 64)
                    cols = pl.ds(hh * 64, 64)
                    qt = q_scr[b, rows, cols]
                    kt = kv_all[:, 0, b, rows, cols]
                    vt = kv_all[:, 1, b, rows, cols]
                    kt = kt.reshape(N_GRP * 64, 64)
                    vt = vt.reshape(N_GRP * 64, 64)
                    s = lax.dot_general(
                        qt, kt, (((1,), (1,)), ((), ())),
                        preferred_element_type=jnp.float32,
                    )
                    m = jnp.max(s, axis=-1, keepdims=True)
                    w = jnp.exp(s - m)
                    w = w / jnp.sum(w, axis=-1, keepdims=True)
                    ctx = jnp.dot(w, vt, preferred_element_type=jnp.float32)
                    ctx_scr[b, rows, cols] = ctx

        out = jnp.dot(ctx_scr[...].reshape(B * S, HD), wo_ref[...],
                      preferred_element_type=jnp.float32)
        out_ref[...] = out.reshape(B, S, D_OUT)

    return pl.pallas_call(
        body,
        out_shape=jax.ShapeDtypeStruct((B, S, D_OUT), jnp.float32),
        in_specs=[pl.BlockSpec(memory_space=pltpu.VMEM)] * 5,
        out_specs=pl.BlockSpec(memory_space=pltpu.VMEM),
        scratch_shapes=[
            pltpu.VMEM((N_GRP, 2, B, S, HD), jnp.float32),
            pltpu.VMEM((B, S, HD), jnp.float32),
            pltpu.VMEM((B, S, HD), jnp.float32),
            pltpu.SemaphoreType.DMA((4, N_GRP - 1)),
            pltpu.SemaphoreType.DMA((4, N_GRP - 1)),
        ],
        compiler_params=pltpu.CompilerParams(collective_id=0),
    )(x, Wq, K2, V2, Wo)


# device time: 69208 ns/iter; 1.5532x vs baseline; 1.5532x over previous
import jax
import jax.numpy as jnp
from jax import lax
from jax.experimental import pallas as pl
from jax.experimental.pallas import tpu as pltpu

N_DEV = 32
N_GRP = 16
B = 2
S = 128
HD = 256
D_OUT = 512


def kernel(x, Wq, K_ext, V_ext, Wo):
    K2 = K_ext.reshape(B, S, HD)
    V2 = V_ext.reshape(B, S, HD)

    def body(x_ref, wq_ref, k_ref, v_ref, wo_ref, out_ref,
             kv_all, q_scr, ctx_scr, send_sems, recv_sems):
        my = lax.axis_index("i")
        right = lax.rem(my + 2, N_DEV)
        left = lax.rem(my + N_DEV - 2, N_DEV)

        barrier_sem = pltpu.get_barrier_semaphore()
        for nbr in (left, right):
            pl.semaphore_signal(
                barrier_sem, inc=1,
                device_id=(nbr,), device_id_type=pl.DeviceIdType.MESH,
            )
        pl.semaphore_wait(barrier_sem, 2)

        kv_all[0, 0] = k_ref[...].astype(jnp.bfloat16)
        kv_all[0, 1] = v_ref[...].astype(jnp.bfloat16)

        xm = x_ref[...].reshape(B * S, D_OUT)
        q = jnp.dot(xm, wq_ref[...], preferred_element_type=jnp.float32)
        q_scr[...] = (q * 0.125).reshape(B, S, HD).astype(jnp.bfloat16)

        chains = [(kv, b) for kv in range(2) for b in range(B)]
        rdmas = [[] for _ in chains]
        for h in range(N_GRP - 1):
            for c, (kv, b) in enumerate(chains):
                if h > 0:
                    rdmas[c][h - 1].wait_recv()
                r = pltpu.make_async_remote_copy(
                    src_ref=kv_all.at[h, kv, b],
                    dst_ref=kv_all.at[h + 1, kv, b],
                    send_sem=send_sems.at[c, h],
                    recv_sem=recv_sems.at[c, h],
                    device_id=(right,),
                    device_id_type=pl.DeviceIdType.MESH,
                )
                r.start()
                rdmas[c].append(r)
        for c in range(len(chains)):
            rdmas[c][-1].wait_recv()
        for h in range(N_GRP - 1):
            for c in range(len(chains)):
                rdmas[c][h].wait_send()

        for b in range(B):
            for hh in range(4):
                for blk in range(2):
                    rows = pl.ds(blk * 64, 64)
                    cols = pl.ds(hh * 64, 64)
                    qt = q_scr[b, rows, cols]
                    kt = kv_all[:, 0, b, rows, cols]
                    vt = kv_all[:, 1, b, rows, cols]
                    kt = kt.reshape(N_GRP * 64, 64)
                    vt = vt.reshape(N_GRP * 64, 64)
                    s = lax.dot_general(
                        qt, kt, (((1,), (1,)), ((), ())),
                        preferred_element_type=jnp.float32,
                    )
                    m = jnp.max(s, axis=-1, keepdims=True)
                    w = jnp.exp(s - m)
                    w = w / jnp.sum(w, axis=-1, keepdims=True)
                    ctx = jnp.dot(w.astype(jnp.bfloat16), vt,
                                  preferred_element_type=jnp.float32)
                    ctx_scr[b, rows, cols] = ctx

        out = jnp.dot(ctx_scr[...].reshape(B * S, HD), wo_ref[...],
                      preferred_element_type=jnp.float32)
        out_ref[...] = out.reshape(B, S, D_OUT)

    return pl.pallas_call(
        body,
        out_shape=jax.ShapeDtypeStruct((B, S, D_OUT), jnp.float32),
        in_specs=[pl.BlockSpec(memory_space=pltpu.VMEM)] * 5,
        out_specs=pl.BlockSpec(memory_space=pltpu.VMEM),
        scratch_shapes=[
            pltpu.VMEM((N_GRP, 2, B, S, HD), jnp.bfloat16),
            pltpu.VMEM((B, S, HD), jnp.bfloat16),
            pltpu.VMEM((B, S, HD), jnp.float32),
            pltpu.SemaphoreType.DMA((4, N_GRP - 1)),
            pltpu.SemaphoreType.DMA((4, N_GRP - 1)),
        ],
        compiler_params=pltpu.CompilerParams(collective_id=0),
    )(x, Wq, K2, V2, Wo)
